# baseline (speedup 1.0000x reference)
import functools

import jax
import jax.numpy as jnp
from jax import lax
from jax.experimental import pallas as pl
from jax.experimental.pallas import tpu as pltpu
from jax.experimental.pallas import tpu_sc as plsc

_B = 16384
_NC = 2
_NS = 16
_NW = _NC * _NS
_BPW = _B // _NW
_HP = _BPW // 2

_FN = 32
_MF = 8
_MG = 128 // _MF

_sc_mesh = plsc.VectorSubcoreMesh(core_axis_name="c", subcore_axis_name="s")


@functools.partial(
    pl.kernel,
    out_type=(
        jax.ShapeDtypeStruct((_B, _FN), jnp.float32),
        jax.ShapeDtypeStruct((_B, _FN), jnp.float32),
        jax.ShapeDtypeStruct((_B, _MF * _MG), jnp.float32),
        jax.ShapeDtypeStruct((_B, _MF * _MG), jnp.float32),
    ),
    mesh=_sc_mesh,
    scratch_types=(
        pltpu.VMEM((_HP,), jnp.int32),
        pltpu.VMEM((_HP,), jnp.int32),
        pltpu.VMEM((_HP,), jnp.int32),
        pltpu.VMEM((_HP,), jnp.int32),
        pltpu.VMEM((_HP, _FN), jnp.float32),
        pltpu.VMEM((_HP, _FN), jnp.float32),
        pltpu.VMEM((_HP, _MF * _MG), jnp.float32),
        pltpu.VMEM((_HP, _MF * _MG), jnp.float32),
        pltpu.SemaphoreType.DMA,
        pltpu.SemaphoreType.DMA,
    ),
    compiler_params=pltpu.CompilerParams(use_tc_tiling_on_sc=False),
)
def _sc_gather(user_hbm, item_hbm, ug_hbm, ig_hbm,
               fnu_tab, fni_tab, mfu_tab, mfi_tab,
               fnu_out, fni_out, mfu_out, mfi_out,
               uidx, iidx, u4, i4, fnu_v, fni_v, mfu_v, mfi_v, gsem, osem):
    wid = lax.axis_index("s") * _NC + lax.axis_index("c")
    for p in range(2):
        base = wid * _BPW + p * _HP
        pltpu.sync_copy(user_hbm.at[pl.ds(base, _HP)], uidx)
        pltpu.sync_copy(item_hbm.at[pl.ds(base, _HP)], iidx)
        pltpu.sync_copy(ug_hbm.at[pl.ds(base, _HP)], u4)
        pltpu.sync_copy(ig_hbm.at[pl.ds(base, _HP)], i4)
        c1 = pltpu.async_copy(fnu_tab.at[uidx], fnu_v, gsem)
        c2 = pltpu.async_copy(fni_tab.at[iidx], fni_v, gsem)
        c3 = pltpu.async_copy(mfu_tab.at[u4], mfu_v, gsem)
        c4 = pltpu.async_copy(mfi_tab.at[i4], mfi_v, gsem)
        c1.wait()
        o1 = pltpu.async_copy(fnu_v, fnu_out.at[pl.ds(base, _HP)], osem)
        c2.wait()
        o2 = pltpu.async_copy(fni_v, fni_out.at[pl.ds(base, _HP)], osem)
        c3.wait()
        o3 = pltpu.async_copy(mfu_v, mfu_out.at[pl.ds(base, _HP)], osem)
        c4.wait()
        o4 = pltpu.async_copy(mfi_v, mfi_out.at[pl.ds(base, _HP)], osem)
        o1.wait()
        o2.wait()
        o3.wait()
        o4.wait()


def _mlp_body(fnu_ref, fni_ref, mfu4_ref, mfi4_ref, usel_ref, isel_ref,
              w1u_ref, w1i_ref, b1_ref, w2_ref, b2_ref, w3_ref, b3_ref,
              womf_ref, woh_ref, bo_ref, out_ref):
    f32 = jnp.float32
    usel = usel_ref[...]
    isel = isel_ref[...]
    mfu4 = mfu4_ref[...]
    mfi4 = mfi4_ref[...]
    mfu = jnp.where(usel == 0, mfu4[:, 0:_MF], 0.0)
    mfi = jnp.where(isel == 0, mfi4[:, 0:_MF], 0.0)
    for j in range(1, _MG):
        mfu += jnp.where(usel == j, mfu4[:, j * _MF:(j + 1) * _MF], 0.0)
        mfi += jnp.where(isel == j, mfi4[:, j * _MF:(j + 1) * _MF], 0.0)
    h = jnp.dot(fnu_ref[...], w1u_ref[...], preferred_element_type=f32)
    h += jnp.dot(fni_ref[...], w1i_ref[...], preferred_element_type=f32)
    h = jnp.maximum(h + b1_ref[...], 0.0)
    h = jnp.maximum(
        jnp.dot(h, w2_ref[...], preferred_element_type=f32) + b2_ref[...], 0.0)
    h = jnp.maximum(
        jnp.dot(h, w3_ref[...], preferred_element_type=f32) + b3_ref[...], 0.0)
    r = jnp.dot(mfu * mfi, womf_ref[...], preferred_element_type=f32)
    r += jnp.dot(h, woh_ref[...], preferred_element_type=f32)
    out_ref[...] = r[:, 0] + bo_ref[0, 0]


def kernel(user, item, mf_emb_user, mf_emb_item, fn_emb_user, fn_emb_item,
           W1, b1, W2, b2, W3, b3, Wo, bo):
    user = user.astype(jnp.int32)
    item = item.astype(jnp.int32)
    fnu, fni, mfu4, mfi4 = _sc_gather(
        user, item, user >> 4, item >> 4,
        fn_emb_user, fn_emb_item,
        mf_emb_user.reshape(-1, _MF * _MG), mf_emb_item.reshape(-1, _MF * _MG))

    blk = 2048
    grid = _B // blk

    def _w(shape):
        return pl.BlockSpec(shape, lambda i: (0, 0))

    out = pl.pallas_call(
        _mlp_body,
        grid=(grid,),
        in_specs=[
            pl.BlockSpec((blk, _FN), lambda i: (i, 0)),
            pl.BlockSpec((blk, _FN), lambda i: (i, 0)),
            pl.BlockSpec((blk, _MF * _MG), lambda i: (i, 0)),
            pl.BlockSpec((blk, _MF * _MG), lambda i: (i, 0)),
            pl.BlockSpec((blk, 1), lambda i: (i, 0)),
            pl.BlockSpec((blk, 1), lambda i: (i, 0)),
            _w((_FN, 64)), _w((_FN, 64)), _w((1, 64)),
            _w((64, 32)), _w((1, 32)),
            _w((32, 16)), _w((1, 16)),
            _w((_MF, 1)), _w((16, 1)), _w((1, 1)),
        ],
        out_specs=pl.BlockSpec((blk,), lambda i: (i,)),
        out_shape=jax.ShapeDtypeStruct((_B,), jnp.float32),
    )(fnu, fni, mfu4, mfi4,
      (user & (_MG - 1)).reshape(_B, 1), (item & (_MG - 1)).reshape(_B, 1),
      W1[:_FN], W1[_FN:], b1.reshape(1, 64),
      W2, b2.reshape(1, 32),
      W3, b3.reshape(1, 16),
      Wo[:_MF], Wo[_MF:], bo.reshape(1, 1))
    return out


# speedup vs baseline: 1.5605x; 1.5605x over previous
import functools

import jax
import jax.numpy as jnp
from jax import lax
from jax.experimental import pallas as pl
from jax.experimental.pallas import tpu as pltpu
from jax.experimental.pallas import tpu_sc as plsc

_B = 16384
_NC = 2
_NS = 16
_NW = _NC * _NS
_BPW = _B // _NW

_FN = 32
_MF = 8

_sc_mesh = plsc.VectorSubcoreMesh(core_axis_name="c", subcore_axis_name="s")


@functools.partial(
    pl.kernel,
    out_type=(
        jax.ShapeDtypeStruct((_B, _FN), jnp.float32),
        jax.ShapeDtypeStruct((_B, _FN), jnp.float32),
        jax.ShapeDtypeStruct((_B, _MF), jnp.float32),
        jax.ShapeDtypeStruct((_B, _MF), jnp.float32),
    ),
    mesh=_sc_mesh,
    scratch_types=(
        pltpu.VMEM((_BPW,), jnp.int32),
        pltpu.VMEM((_BPW,), jnp.int32),
        pltpu.VMEM((_BPW, _FN), jnp.float32),
        pltpu.VMEM((_BPW, _FN), jnp.float32),
        pltpu.VMEM((_BPW, _MF), jnp.float32),
        pltpu.VMEM((_BPW, _MF), jnp.float32),
        pltpu.SemaphoreType.DMA,
        pltpu.SemaphoreType.DMA,
    ),
    compiler_params=pltpu.CompilerParams(use_tc_tiling_on_sc=False),
)
def _sc_gather(user_hbm, item_hbm, fnu_tab, fni_tab, mfu_tab, mfi_tab,
               fnu_out, fni_out, mfu_out, mfi_out,
               uidx, iidx, fnu_v, fni_v, mfu_v, mfi_v, gsem, osem):
    wid = lax.axis_index("s") * _NC + lax.axis_index("c")
    base = wid * _BPW
    pltpu.sync_copy(user_hbm.at[pl.ds(base, _BPW)], uidx)
    pltpu.sync_copy(item_hbm.at[pl.ds(base, _BPW)], iidx)
    c1 = pltpu.async_copy(fnu_tab.at[uidx], fnu_v, gsem)
    c2 = pltpu.async_copy(fni_tab.at[iidx], fni_v, gsem)
    c3 = pltpu.async_copy(mfu_tab.at[uidx], mfu_v, gsem)
    c4 = pltpu.async_copy(mfi_tab.at[iidx], mfi_v, gsem)
    c1.wait()
    o1 = pltpu.async_copy(fnu_v, fnu_out.at[pl.ds(base, _BPW)], osem)
    c2.wait()
    o2 = pltpu.async_copy(fni_v, fni_out.at[pl.ds(base, _BPW)], osem)
    c3.wait()
    o3 = pltpu.async_copy(mfu_v, mfu_out.at[pl.ds(base, _BPW)], osem)
    c4.wait()
    o4 = pltpu.async_copy(mfi_v, mfi_out.at[pl.ds(base, _BPW)], osem)
    o1.wait()
    o2.wait()
    o3.wait()
    o4.wait()


def _mlp_body(fnu_ref, fni_ref, mfu_ref, mfi_ref, w1u_ref, w1i_ref, b1_ref,
              w2_ref, b2_ref, w3_ref, b3_ref, womf_ref, woh_ref, bo_ref,
              out_ref):
    f32 = jnp.float32
    h = jnp.dot(fnu_ref[...], w1u_ref[...], preferred_element_type=f32)
    h += jnp.dot(fni_ref[...], w1i_ref[...], preferred_element_type=f32)
    h = jnp.maximum(h + b1_ref[...], 0.0)
    h = jnp.maximum(
        jnp.dot(h, w2_ref[...], preferred_element_type=f32) + b2_ref[...], 0.0)
    h = jnp.maximum(
        jnp.dot(h, w3_ref[...], preferred_element_type=f32) + b3_ref[...], 0.0)
    r = jnp.dot(mfu_ref[...] * mfi_ref[...], womf_ref[...],
                preferred_element_type=f32)
    r += jnp.dot(h, woh_ref[...], preferred_element_type=f32)
    out_ref[...] = r[:, 0] + bo_ref[0, 0]


def kernel(user, item, mf_emb_user, mf_emb_item, fn_emb_user, fn_emb_item,
           W1, b1, W2, b2, W3, b3, Wo, bo):
    fnu, fni, mfu, mfi = _sc_gather(
        user.astype(jnp.int32), item.astype(jnp.int32),
        fn_emb_user, fn_emb_item, mf_emb_user, mf_emb_item)

    blk = 2048
    grid = _B // blk

    def _w(shape):
        return pl.BlockSpec(shape, lambda i: (0, 0))

    out = pl.pallas_call(
        _mlp_body,
        grid=(grid,),
        in_specs=[
            pl.BlockSpec((blk, _FN), lambda i: (i, 0)),
            pl.BlockSpec((blk, _FN), lambda i: (i, 0)),
            pl.BlockSpec((blk, _MF), lambda i: (i, 0)),
            pl.BlockSpec((blk, _MF), lambda i: (i, 0)),
            _w((_FN, 64)), _w((_FN, 64)), _w((1, 64)),
            _w((64, 32)), _w((1, 32)),
            _w((32, 16)), _w((1, 16)),
            _w((_MF, 1)), _w((16, 1)), _w((1, 1)),
        ],
        out_specs=pl.BlockSpec((blk,), lambda i: (i,)),
        out_shape=jax.ShapeDtypeStruct((_B,), jnp.float32),
    )(fnu, fni, mfu, mfi,
      W1[:_FN], W1[_FN:], b1.reshape(1, 64),
      W2, b2.reshape(1, 32),
      W3, b3.reshape(1, 16),
      Wo[:_MF], Wo[_MF:], bo.reshape(1, 1))
    return out
